# Initial kernel scaffold; baseline (speedup 1.0000x reference)
import jax
import jax.numpy as jnp
from jax.experimental import pallas as pl


def kernel(hidden_states, gate_w, W_gate, W_up, W_down):
    raise NotImplementedError("write your pallas kernel here")


# speedup vs baseline: 2.3657x; 2.3657x over previous
import functools

import jax
import jax.numpy as jnp
from jax.experimental import pallas as pl
from jax.experimental.pallas import tpu as pltpu

E = 8
TOP_K = 2
D_MODEL = 768
D_FF = 384
T = 2048
BLK_T = 256


def _moe_block(x_ref, gate_ref, wg_ref, wu_ref, wd_ref, y_ref):
    xb = x_ref[...]
    logits = jax.lax.dot_general(
        xb, gate_ref[...], (((1,), (1,)), ((), ())),
        preferred_element_type=jnp.float32)
    m = jnp.max(logits, axis=1, keepdims=True)
    ex = jnp.exp(logits - m)
    s = ex / jnp.sum(ex, axis=1, keepdims=True)
    idx = jax.lax.broadcasted_iota(jnp.int32, (BLK_T, E), 1)
    v1 = jnp.max(s, axis=1, keepdims=True)
    i1 = jnp.min(jnp.where(s == v1, idx, E), axis=1, keepdims=True)
    one1 = idx == i1
    s2 = jnp.where(one1, -jnp.inf, s)
    v2 = jnp.max(s2, axis=1, keepdims=True)
    i2 = jnp.min(jnp.where(s2 == v2, idx, E), axis=1, keepdims=True)
    one2 = idx == i2
    wfull = (jnp.where(one1, v1, 0.0) + jnp.where(one2, v2, 0.0)) / (v1 + v2)

    acc = jnp.zeros((BLK_T, D_MODEL), dtype=jnp.float32)
    for e in range(E):
        g = jax.lax.dot_general(
            xb, wg_ref[e], (((1,), (1,)), ((), ())),
            preferred_element_type=jnp.float32)
        u = jax.lax.dot_general(
            xb, wu_ref[e], (((1,), (1,)), ((), ())),
            preferred_element_type=jnp.float32)
        h = (g / (1.0 + jnp.exp(-g))) * u
        o = jax.lax.dot_general(
            h, wd_ref[e], (((1,), (1,)), ((), ())),
            preferred_element_type=jnp.float32)
        acc = acc + wfull[:, e:e + 1] * o
    y_ref[...] = acc


@functools.partial(jax.jit, static_argnames=())
def _moe(x, gate_w, W_gate, W_up, W_down):
    grid = (T // BLK_T,)
    return pl.pallas_call(
        _moe_block,
        grid=grid,
        in_specs=[
            pl.BlockSpec((BLK_T, D_MODEL), lambda i: (i, 0)),
            pl.BlockSpec((E, D_MODEL), lambda i: (0, 0)),
            pl.BlockSpec((E, D_FF, D_MODEL), lambda i: (0, 0, 0)),
            pl.BlockSpec((E, D_FF, D_MODEL), lambda i: (0, 0, 0)),
            pl.BlockSpec((E, D_MODEL, D_FF), lambda i: (0, 0, 0)),
        ],
        out_specs=pl.BlockSpec((BLK_T, D_MODEL), lambda i: (i, 0)),
        out_shape=jax.ShapeDtypeStruct((T, D_MODEL), jnp.float32),
    )(x, gate_w, W_gate, W_up, W_down)


def kernel(hidden_states, gate_w, W_gate, W_up, W_down):
    orig_shape = hidden_states.shape
    x = hidden_states.reshape(-1, orig_shape[-1])
    y = _moe(x, gate_w, W_gate, W_up, W_down)
    return y.reshape(orig_shape)
